# Initial kernel scaffold; baseline (speedup 1.0000x reference)
import jax
import jax.numpy as jnp
from jax.experimental import pallas as pl


def kernel(tensor, idx):
    raise NotImplementedError("write your pallas kernel here")


# speedup vs baseline: 2.0697x; 2.0697x over previous
import functools

import jax
import jax.numpy as jnp
from jax import lax
from jax.experimental import pallas as pl
from jax.experimental.pallas import tpu as pltpu
from jax.experimental.pallas import tpu_sc as plsc

_NC = 2
_NS = 16
_NW = _NC * _NS

_CHUNK = 40


def _make_gather(V, D, B):
    b_per_w = B // _NW
    n_chunks = b_per_w // _CHUNK
    mesh = plsc.VectorSubcoreMesh(core_axis_name="c", subcore_axis_name="s")

    @functools.partial(
        pl.kernel,
        mesh=mesh,
        out_type=jax.ShapeDtypeStruct((B, D), jnp.float32),
        scratch_types=[
            pltpu.VMEM((n_chunks, _CHUNK), jnp.int32),
            pltpu.VMEM((_CHUNK, D), jnp.float32),
            pltpu.SemaphoreType.DMA,
        ],
    )
    def gather_kernel(table_hbm, idx_hbm, out_hbm, idx_v, rows_v, sem):
        wid = lax.axis_index("s") * _NC + lax.axis_index("c")
        base = wid * b_per_w
        pltpu.sync_copy(idx_hbm.at[wid], idx_v)

        def body(j, carry):
            pltpu.async_copy(table_hbm.at[idx_v.at[j]], rows_v, sem).wait()
            pltpu.sync_copy(rows_v, out_hbm.at[pl.ds(base + j * _CHUNK, _CHUNK)])
            return carry

        lax.fori_loop(0, n_chunks, body, 0)

    return gather_kernel


def kernel(tensor, idx):
    V, D = tensor.shape
    (B,) = idx.shape
    b_per_w = B // _NW
    idx3 = idx.reshape(_NW, b_per_w // _CHUNK, _CHUNK)
    return _make_gather(V, D, B)(tensor, idx3)


# speedup vs baseline: 2.9529x; 1.4268x over previous
import functools

import jax
import jax.numpy as jnp
from jax import lax
from jax.experimental import pallas as pl
from jax.experimental.pallas import tpu as pltpu
from jax.experimental.pallas import tpu_sc as plsc

_NC = 2
_NS = 16
_NW = _NC * _NS

_CHUNK = 40


def _make_gather(V, D, B):
    b_per_w = B // _NW
    n_chunks = b_per_w // _CHUNK
    mesh = plsc.VectorSubcoreMesh(core_axis_name="c", subcore_axis_name="s")

    @functools.partial(
        pl.kernel,
        mesh=mesh,
        out_type=jax.ShapeDtypeStruct((B, D), jnp.float32),
        scratch_types=[
            pltpu.VMEM((n_chunks, _CHUNK), jnp.int32),
            pltpu.VMEM((_CHUNK, D), jnp.float32),
            pltpu.VMEM((_CHUNK, D), jnp.float32),
            pltpu.SemaphoreType.DMA,
            pltpu.SemaphoreType.DMA,
            pltpu.SemaphoreType.DMA,
            pltpu.SemaphoreType.DMA,
        ],
    )
    def gather_kernel(table_hbm, idx_hbm, out_hbm, idx_v, buf0, buf1,
                      g0, g1, s0, s1):
        wid = lax.axis_index("s") * _NC + lax.axis_index("c")
        base = wid * b_per_w
        pltpu.sync_copy(idx_hbm.at[wid], idx_v)

        def gather_start(j, buf, sem):
            pltpu.async_copy(table_hbm.at[idx_v.at[j]], buf, sem)

        def gather_wait(buf, sem):
            pltpu.make_async_copy(table_hbm.at[idx_v.at[0]], buf, sem).wait()

        def scatter_start(j, buf, sem):
            dst = out_hbm.at[pl.ds(base + j * _CHUNK, _CHUNK)]
            pltpu.async_copy(buf, dst, sem)

        def scatter_wait(buf, sem):
            dst = out_hbm.at[pl.ds(base, _CHUNK)]
            pltpu.make_async_copy(buf, dst, sem).wait()

        gather_start(0, buf0, g0)
        gather_start(1, buf1, g1)

        def body(i, carry):
            g = 2 * i
            gather_wait(buf0, g0)
            scatter_start(g, buf0, s0)
            gather_wait(buf1, g1)
            scatter_start(g + 1, buf1, s1)
            scatter_wait(buf0, s0)
            gather_start(g + 2, buf0, g0)
            scatter_wait(buf1, s1)
            gather_start(g + 3, buf1, g1)
            return carry

        lax.fori_loop(0, (n_chunks - 3) // 2, body, 0)

        g = n_chunks - 3
        gather_wait(buf0, g0)
        scatter_start(g, buf0, s0)
        gather_wait(buf1, g1)
        scatter_start(g + 1, buf1, s1)
        scatter_wait(buf0, s0)
        gather_start(g + 2, buf0, g0)
        gather_wait(buf0, g0)
        scatter_start(g + 2, buf0, s0)
        scatter_wait(buf0, s0)
        scatter_wait(buf1, s1)

    return gather_kernel


def kernel(tensor, idx):
    V, D = tensor.shape
    (B,) = idx.shape
    b_per_w = B // _NW
    idx3 = idx.reshape(_NW, b_per_w // _CHUNK, _CHUNK)
    return _make_gather(V, D, B)(tensor, idx3)


# speedup vs baseline: 3.4643x; 1.1732x over previous
import functools

import jax
import jax.numpy as jnp
from jax import lax
from jax.experimental import pallas as pl
from jax.experimental.pallas import tpu as pltpu
from jax.experimental.pallas import tpu_sc as plsc

_NC = 2
_NS = 16
_NW = _NC * _NS

_CHUNK = 40
_NBUF = 5


def _make_gather(V, D, B):
    b_per_w = B // _NW
    n_chunks = b_per_w // _CHUNK
    mesh = plsc.VectorSubcoreMesh(core_axis_name="c", subcore_axis_name="s")

    @functools.partial(
        pl.kernel,
        mesh=mesh,
        out_type=jax.ShapeDtypeStruct((B, D), jnp.float32),
        scratch_types=[
            pltpu.VMEM((n_chunks, _CHUNK), jnp.int32),
        ] + [pltpu.VMEM((_CHUNK, D), jnp.float32)] * _NBUF
          + [pltpu.SemaphoreType.DMA] * (2 * _NBUF),
    )
    def gather_kernel(table_hbm, idx_hbm, out_hbm, idx_v, *rest):
        bufs = rest[:_NBUF]
        gsems = rest[_NBUF:2 * _NBUF]
        ssems = rest[2 * _NBUF:]
        wid = lax.axis_index("s") * _NC + lax.axis_index("c")
        base = wid * b_per_w
        pltpu.sync_copy(idx_hbm.at[wid], idx_v)

        def gather_start(j, b):
            pltpu.async_copy(table_hbm.at[idx_v.at[j]], bufs[b], gsems[b])

        def gather_wait(b):
            pltpu.make_async_copy(
                table_hbm.at[idx_v.at[0]], bufs[b], gsems[b]).wait()

        def scatter_start(j, b):
            dst = out_hbm.at[pl.ds(base + j * _CHUNK, _CHUNK)]
            pltpu.async_copy(bufs[b], dst, ssems[b])

        def scatter_wait(b):
            dst = out_hbm.at[pl.ds(base, _CHUNK)]
            pltpu.make_async_copy(bufs[b], dst, ssems[b]).wait()

        for b in range(_NBUF):
            gather_start(b, b)

        def body(i, carry):
            g = _NBUF * i
            for b in range(_NBUF):
                gather_wait(b)
                scatter_start(g + b, b)
            for b in range(_NBUF):
                scatter_wait(b)
                gather_start(g + _NBUF + b, b)
            return carry

        lax.fori_loop(0, n_chunks // _NBUF - 1, body, 0)

        g = n_chunks - _NBUF
        for b in range(_NBUF):
            gather_wait(b)
            scatter_start(g + b, b)
        for b in range(_NBUF):
            scatter_wait(b)

    return gather_kernel


def kernel(tensor, idx):
    V, D = tensor.shape
    (B,) = idx.shape
    b_per_w = B // _NW
    idx3 = idx.reshape(_NW, b_per_w // _CHUNK, _CHUNK)
    return _make_gather(V, D, B)(tensor, idx3)
